# initial kernel scaffold (unmeasured)
import jax
import jax.numpy as jnp
from jax import lax
from jax.experimental import pallas as pl
from jax.experimental.pallas import tpu as pltpu

N_DEV = 32


def kernel(x, w_mat):
    m_total, k_shard = x.shape
    k_total, n = w_mat.shape
    m_per = m_total // N_DEV
    assert k_total == k_shard * N_DEV

    def body(x_ref, w_ref, out_ref, xg_ref, acc_ref, amax_ref,
             a2a_send_sems, a2a_recv_sems, am_send_sems, am_recv_sems):
        my = lax.axis_index("i")

        barrier_sem = pltpu.get_barrier_semaphore()
        for p in range(N_DEV):
            @pl.when(p != my)
            def _():
                pl.semaphore_signal(
                    barrier_sem, inc=1,
                    device_id=(p,), device_id_type=pl.DeviceIdType.MESH,
                )
        pl.semaphore_wait(barrier_sem, N_DEV - 1)

        for p in range(N_DEV):
            @pl.when(p != my)
            def _():
                rdma = pltpu.make_async_remote_copy(
                    src_ref=x_ref.at[pl.ds(p * m_per, m_per), :],
                    dst_ref=xg_ref.at[my],
                    send_sem=a2a_send_sems.at[p],
                    recv_sem=a2a_recv_sems.at[my],
                    device_id=(p,),
                    device_id_type=pl.DeviceIdType.MESH,
                )
                rdma.start()

        acc_ref[...] = jnp.dot(
            x_ref[pl.ds(my * m_per, m_per), :],
            w_ref[pl.ds(my * k_shard, k_shard), :],
            preferred_element_type=jnp.float32,
        )

        for p in range(N_DEV):
            @pl.when(p != my)
            def _():
                recv = pltpu.make_async_remote_copy(
                    src_ref=x_ref.at[pl.ds(0, m_per), :],
                    dst_ref=xg_ref.at[p],
                    send_sem=a2a_send_sems.at[p],
                    recv_sem=a2a_recv_sems.at[p],
                    device_id=(p,),
                    device_id_type=pl.DeviceIdType.MESH,
                )
                recv.wait_recv()
                acc_ref[...] += jnp.dot(
                    xg_ref[p],
                    w_ref[p * k_shard:(p + 1) * k_shard, :],
                    preferred_element_type=jnp.float32,
                )

        for p in range(N_DEV):
            @pl.when(p != my)
            def _():
                snd = pltpu.make_async_remote_copy(
                    src_ref=x_ref.at[pl.ds(p * m_per, m_per), :],
                    dst_ref=xg_ref.at[p],
                    send_sem=a2a_send_sems.at[p],
                    recv_sem=a2a_recv_sems.at[p],
                    device_id=(p,),
                    device_id_type=pl.DeviceIdType.MESH,
                )
                snd.wait_send()

        y = jnp.maximum(acc_ref[...], 0.0)
        local_amax = jnp.max(y)
        amax_ref[pl.ds(my, 1), :] = jnp.full((1, 128), local_amax, jnp.float32)

        for p in range(N_DEV):
            @pl.when(p != my)
            def _():
                rdma = pltpu.make_async_remote_copy(
                    src_ref=amax_ref.at[pl.ds(my, 1), :],
                    dst_ref=amax_ref.at[pl.ds(my, 1), :],
                    send_sem=am_send_sems.at[p],
                    recv_sem=am_recv_sems.at[my],
                    device_id=(p,),
                    device_id_type=pl.DeviceIdType.MESH,
                )
                rdma.start()
        for p in range(N_DEV):
            @pl.when(p != my)
            def _():
                recv = pltpu.make_async_remote_copy(
                    src_ref=amax_ref.at[pl.ds(p, 1), :],
                    dst_ref=amax_ref.at[pl.ds(p, 1), :],
                    send_sem=am_send_sems.at[p],
                    recv_sem=am_recv_sems.at[p],
                    device_id=(p,),
                    device_id_type=pl.DeviceIdType.MESH,
                )
                recv.wait_recv()
        for p in range(N_DEV):
            @pl.when(p != my)
            def _():
                snd = pltpu.make_async_remote_copy(
                    src_ref=amax_ref.at[pl.ds(my, 1), :],
                    dst_ref=amax_ref.at[pl.ds(my, 1), :],
                    send_sem=am_send_sems.at[p],
                    recv_sem=am_recv_sems.at[p],
                    device_id=(p,),
                    device_id_type=pl.DeviceIdType.MESH,
                )
                snd.wait_send()

        g_amax = jnp.max(amax_ref[...])
        scale = g_amax / 448.0
        q = jnp.minimum(y / scale, 448.0)
        q8 = q.astype(jnp.float8_e4m3fn).astype(jnp.float32)
        out_ref[...] = q8 * scale

    return pl.pallas_call(
        body,
        out_shape=jax.ShapeDtypeStruct((m_per, n), jnp.float32),
        in_specs=[
            pl.BlockSpec(memory_space=pltpu.VMEM),
            pl.BlockSpec(memory_space=pltpu.VMEM),
        ],
        out_specs=pl.BlockSpec(memory_space=pltpu.VMEM),
        scratch_shapes=[
            pltpu.VMEM((N_DEV, m_per, k_shard), jnp.float32),
            pltpu.VMEM((m_per, n), jnp.float32),
            pltpu.VMEM((N_DEV, 128), jnp.float32),
            pltpu.SemaphoreType.DMA((N_DEV,)),
            pltpu.SemaphoreType.DMA((N_DEV,)),
            pltpu.SemaphoreType.DMA((N_DEV,)),
            pltpu.SemaphoreType.DMA((N_DEV,)),
        ],
        compiler_params=pltpu.CompilerParams(collective_id=0),
    )(x, w_mat)


# baseline (device time: 70539 ns/iter reference)
import jax
import jax.numpy as jnp
from jax import lax
from jax.experimental import pallas as pl
from jax.experimental.pallas import tpu as pltpu

N_DEV = 32


def kernel(x, w_mat):
    m_total, k_shard = x.shape
    k_total, n = w_mat.shape
    m_per = m_total // N_DEV
    assert k_total == k_shard * N_DEV

    def body(x_ref, w_ref, out_ref, xg_ref, acc_ref, amax_ref,
             a2a_send_sems, a2a_recv_sems, am_send_sems, am_recv_sems):
        my = lax.axis_index("i")

        barrier_sem = pltpu.get_barrier_semaphore()
        for p in range(N_DEV):
            @pl.when(p != my)
            def _():
                pl.semaphore_signal(
                    barrier_sem, inc=1,
                    device_id=(p,), device_id_type=pl.DeviceIdType.MESH,
                )
        pl.semaphore_wait(barrier_sem, N_DEV - 1)

        for p in range(N_DEV):
            @pl.when(p != my)
            def _():
                rdma = pltpu.make_async_remote_copy(
                    src_ref=x_ref.at[pl.ds(p * m_per, m_per), :],
                    dst_ref=xg_ref.at[my],
                    send_sem=a2a_send_sems.at[p],
                    recv_sem=a2a_recv_sems.at[my],
                    device_id=(p,),
                    device_id_type=pl.DeviceIdType.MESH,
                )
                rdma.start()

        acc_ref[...] = jnp.dot(
            x_ref[pl.ds(my * m_per, m_per), :],
            w_ref[pl.ds(my * k_shard, k_shard), :],
            preferred_element_type=jnp.float32,
        )

        for p in range(N_DEV):
            @pl.when(p != my)
            def _():
                recv = pltpu.make_async_remote_copy(
                    src_ref=x_ref.at[pl.ds(0, m_per), :],
                    dst_ref=xg_ref.at[p],
                    send_sem=a2a_send_sems.at[p],
                    recv_sem=a2a_recv_sems.at[p],
                    device_id=(p,),
                    device_id_type=pl.DeviceIdType.MESH,
                )
                recv.wait_recv()
                acc_ref[...] += jnp.dot(
                    xg_ref[p],
                    w_ref[p * k_shard:(p + 1) * k_shard, :],
                    preferred_element_type=jnp.float32,
                )

        for p in range(N_DEV):
            @pl.when(p != my)
            def _():
                snd = pltpu.make_async_remote_copy(
                    src_ref=x_ref.at[pl.ds(p * m_per, m_per), :],
                    dst_ref=xg_ref.at[p],
                    send_sem=a2a_send_sems.at[p],
                    recv_sem=a2a_recv_sems.at[p],
                    device_id=(p,),
                    device_id_type=pl.DeviceIdType.MESH,
                )
                snd.wait_send()

        y = jnp.maximum(acc_ref[...], 0.0)
        local_amax = jnp.max(y)
        amax_ref[pl.ds(my, 1), :] = jnp.full((1, 128), local_amax, jnp.float32)

        for p in range(N_DEV):
            @pl.when(p != my)
            def _():
                rdma = pltpu.make_async_remote_copy(
                    src_ref=amax_ref.at[pl.ds(my, 1), :],
                    dst_ref=amax_ref.at[pl.ds(my, 1), :],
                    send_sem=am_send_sems.at[p],
                    recv_sem=am_recv_sems.at[my],
                    device_id=(p,),
                    device_id_type=pl.DeviceIdType.MESH,
                )
                rdma.start()
        for p in range(N_DEV):
            @pl.when(p != my)
            def _():
                recv = pltpu.make_async_remote_copy(
                    src_ref=amax_ref.at[pl.ds(p, 1), :],
                    dst_ref=amax_ref.at[pl.ds(p, 1), :],
                    send_sem=am_send_sems.at[p],
                    recv_sem=am_recv_sems.at[p],
                    device_id=(p,),
                    device_id_type=pl.DeviceIdType.MESH,
                )
                recv.wait_recv()
        for p in range(N_DEV):
            @pl.when(p != my)
            def _():
                snd = pltpu.make_async_remote_copy(
                    src_ref=amax_ref.at[pl.ds(my, 1), :],
                    dst_ref=amax_ref.at[pl.ds(my, 1), :],
                    send_sem=am_send_sems.at[p],
                    recv_sem=am_recv_sems.at[p],
                    device_id=(p,),
                    device_id_type=pl.DeviceIdType.MESH,
                )
                snd.wait_send()

        g_amax = jnp.max(amax_ref[...])
        scale = g_amax / 448.0
        q = jnp.minimum(y / scale, 448.0)
        q8 = q.astype(jnp.float8_e4m3fn).astype(jnp.float32)
        out_ref[...] = q8 * scale

    return pl.pallas_call(
        body,
        out_shape=jax.ShapeDtypeStruct((m_per, n), jnp.float32),
        in_specs=[
            pl.BlockSpec(memory_space=pltpu.VMEM),
            pl.BlockSpec(memory_space=pltpu.VMEM),
        ],
        out_specs=pl.BlockSpec(memory_space=pltpu.VMEM),
        scratch_shapes=[
            pltpu.VMEM((N_DEV, m_per, k_shard), jnp.float32),
            pltpu.VMEM((m_per, n), jnp.float32),
            pltpu.VMEM((N_DEV, 128), jnp.float32),
            pltpu.SemaphoreType.DMA((N_DEV,)),
            pltpu.SemaphoreType.DMA((N_DEV,)),
            pltpu.SemaphoreType.DMA((N_DEV,)),
            pltpu.SemaphoreType.DMA((N_DEV,)),
        ],
        compiler_params=pltpu.CompilerParams(
            collective_id=0,
            vmem_limit_bytes=100 * 1024 * 1024,
        ),
    )(x, w_mat)


# device time: 66420 ns/iter; 1.0620x vs baseline; 1.0620x over previous
import jax
import jax.numpy as jnp
from jax import lax
from jax.experimental import pallas as pl
from jax.experimental.pallas import tpu as pltpu

N_DEV = 32


def kernel(x, w_mat):
    m_total, k_shard = x.shape
    k_total, n = w_mat.shape
    m_per = m_total // N_DEV
    assert k_total == k_shard * N_DEV

    def body(x_ref, w_ref, out_ref, xg_ref, amax_ref,
             a2a_send_sems, a2a_recv_sems, am_send_sems, am_recv_sems):
        my = lax.axis_index("i")

        barrier_sem = pltpu.get_barrier_semaphore()
        for p in range(N_DEV):
            @pl.when(p != my)
            def _():
                pl.semaphore_signal(
                    barrier_sem, inc=1,
                    device_id=(p,), device_id_type=pl.DeviceIdType.MESH,
                )
        pl.semaphore_wait(barrier_sem, N_DEV - 1)

        for p in range(N_DEV):
            @pl.when(p != my)
            def _():
                rdma = pltpu.make_async_remote_copy(
                    src_ref=x_ref.at[pl.ds(p * m_per, m_per), :],
                    dst_ref=xg_ref.at[:, pl.ds(my * k_shard, k_shard)],
                    send_sem=a2a_send_sems.at[p],
                    recv_sem=a2a_recv_sems.at[my],
                    device_id=(p,),
                    device_id_type=pl.DeviceIdType.MESH,
                )
                rdma.start()

        xg_ref[:, pl.ds(my * k_shard, k_shard)] = x_ref[pl.ds(my * m_per, m_per), :]

        G = 4
        k_grp = G * k_shard
        acc = None
        for g in range(N_DEV // G):
            for p in range(g * G, (g + 1) * G):
                @pl.when(p != my)
                def _():
                    recv = pltpu.make_async_remote_copy(
                        src_ref=x_ref.at[pl.ds(0, m_per), :],
                        dst_ref=xg_ref.at[:, pl.ds(p * k_shard, k_shard)],
                        send_sem=a2a_send_sems.at[p],
                        recv_sem=a2a_recv_sems.at[p],
                        device_id=(p,),
                        device_id_type=pl.DeviceIdType.MESH,
                    )
                    recv.wait_recv()
            part = jnp.dot(
                xg_ref[:, g * k_grp:(g + 1) * k_grp],
                w_ref[g * k_grp:(g + 1) * k_grp, :],
                preferred_element_type=jnp.float32,
            )
            acc = part if acc is None else acc + part

        for p in range(N_DEV):
            @pl.when(p != my)
            def _():
                snd = pltpu.make_async_remote_copy(
                    src_ref=x_ref.at[pl.ds(p * m_per, m_per), :],
                    dst_ref=xg_ref.at[:, pl.ds(p * k_shard, k_shard)],
                    send_sem=a2a_send_sems.at[p],
                    recv_sem=a2a_recv_sems.at[p],
                    device_id=(p,),
                    device_id_type=pl.DeviceIdType.MESH,
                )
                snd.wait_send()

        y = jnp.maximum(acc, 0.0)
        local_amax = jnp.max(y)
        amax_ref[pl.ds(my, 1), :] = jnp.full((1, 128), local_amax, jnp.float32)

        for p in range(N_DEV):
            @pl.when(p != my)
            def _():
                rdma = pltpu.make_async_remote_copy(
                    src_ref=amax_ref.at[pl.ds(my, 1), :],
                    dst_ref=amax_ref.at[pl.ds(my, 1), :],
                    send_sem=am_send_sems.at[p],
                    recv_sem=am_recv_sems.at[my],
                    device_id=(p,),
                    device_id_type=pl.DeviceIdType.MESH,
                )
                rdma.start()
        for p in range(N_DEV):
            @pl.when(p != my)
            def _():
                recv = pltpu.make_async_remote_copy(
                    src_ref=amax_ref.at[pl.ds(p, 1), :],
                    dst_ref=amax_ref.at[pl.ds(p, 1), :],
                    send_sem=am_send_sems.at[p],
                    recv_sem=am_recv_sems.at[p],
                    device_id=(p,),
                    device_id_type=pl.DeviceIdType.MESH,
                )
                recv.wait_recv()
        for p in range(N_DEV):
            @pl.when(p != my)
            def _():
                snd = pltpu.make_async_remote_copy(
                    src_ref=amax_ref.at[pl.ds(my, 1), :],
                    dst_ref=amax_ref.at[pl.ds(my, 1), :],
                    send_sem=am_send_sems.at[p],
                    recv_sem=am_recv_sems.at[p],
                    device_id=(p,),
                    device_id_type=pl.DeviceIdType.MESH,
                )
                snd.wait_send()

        g_amax = jnp.max(amax_ref[...])
        scale = g_amax / 448.0
        q = jnp.minimum(y / scale, 448.0)
        q8 = q.astype(jnp.float8_e4m3fn).astype(jnp.float32)
        out_ref[...] = q8 * scale

    return pl.pallas_call(
        body,
        out_shape=jax.ShapeDtypeStruct((m_per, n), jnp.float32),
        in_specs=[
            pl.BlockSpec(memory_space=pltpu.VMEM),
            pl.BlockSpec(memory_space=pltpu.VMEM),
        ],
        out_specs=pl.BlockSpec(memory_space=pltpu.VMEM),
        scratch_shapes=[
            pltpu.VMEM((m_per, k_total), jnp.float32),
            pltpu.VMEM((N_DEV, 128), jnp.float32),
            pltpu.SemaphoreType.DMA((N_DEV,)),
            pltpu.SemaphoreType.DMA((N_DEV,)),
            pltpu.SemaphoreType.DMA((N_DEV,)),
            pltpu.SemaphoreType.DMA((N_DEV,)),
        ],
        compiler_params=pltpu.CompilerParams(
            collective_id=0,
            vmem_limit_bytes=100 * 1024 * 1024,
        ),
    )(x, w_mat)


# device time: 57430 ns/iter; 1.2283x vs baseline; 1.1565x over previous
import jax
import jax.numpy as jnp
from jax import lax
from jax.experimental import pallas as pl
from jax.experimental.pallas import tpu as pltpu

N_DEV = 32


def kernel(x, w_mat):
    m_total, k_shard = x.shape
    k_total, n = w_mat.shape
    m_per = m_total // N_DEV
    assert k_total == k_shard * N_DEV

    def body(x_ref, w_ref, out_ref, xg_ref, amax_ref,
             a2a_send_sems, a2a_recv_sems, am_send_sems, am_recv_sems):
        my = lax.axis_index("i")

        barrier_sem = pltpu.get_barrier_semaphore()
        for p in range(N_DEV):
            @pl.when(p != my)
            def _():
                pl.semaphore_signal(
                    barrier_sem, inc=1,
                    device_id=(p,), device_id_type=pl.DeviceIdType.MESH,
                )
        pl.semaphore_wait(barrier_sem, N_DEV - 1)

        for d in range(1, N_DEV):
            dst = (my + d) % N_DEV
            rdma = pltpu.make_async_remote_copy(
                src_ref=x_ref.at[pl.ds(dst * m_per, m_per), :],
                dst_ref=xg_ref.at[my],
                send_sem=a2a_send_sems.at[dst],
                recv_sem=a2a_recv_sems.at[my],
                device_id=(dst,),
                device_id_type=pl.DeviceIdType.MESH,
            )
            rdma.start()

        acc = jnp.dot(
            x_ref[pl.ds(my * m_per, m_per), :],
            w_ref[pl.ds(my * k_shard, k_shard), :],
            preferred_element_type=jnp.float32,
        )

        for d in range(1, N_DEV):
            s = (my - d) % N_DEV
            recv = pltpu.make_async_remote_copy(
                src_ref=x_ref.at[pl.ds(0, m_per), :],
                dst_ref=xg_ref.at[s],
                send_sem=a2a_send_sems.at[s],
                recv_sem=a2a_recv_sems.at[s],
                device_id=(s,),
                device_id_type=pl.DeviceIdType.MESH,
            )
            recv.wait_recv()
            acc = acc + jnp.dot(
                xg_ref[s],
                w_ref[pl.ds(s * k_shard, k_shard), :],
                preferred_element_type=jnp.float32,
            )

        for d in range(1, N_DEV):
            dst = (my + d) % N_DEV
            snd = pltpu.make_async_remote_copy(
                src_ref=x_ref.at[pl.ds(dst * m_per, m_per), :],
                dst_ref=xg_ref.at[my],
                send_sem=a2a_send_sems.at[dst],
                recv_sem=a2a_recv_sems.at[my],
                device_id=(dst,),
                device_id_type=pl.DeviceIdType.MESH,
            )
            snd.wait_send()

        y = jnp.maximum(acc, 0.0)
        local_amax = jnp.max(y)
        amax_ref[pl.ds(my, 1), :] = jnp.full((1, 128), local_amax, jnp.float32)

        for d in range(1, N_DEV):
            dst = (my + d) % N_DEV
            rdma = pltpu.make_async_remote_copy(
                src_ref=amax_ref.at[pl.ds(my, 1), :],
                dst_ref=amax_ref.at[pl.ds(my, 1), :],
                send_sem=am_send_sems.at[dst],
                recv_sem=am_recv_sems.at[my],
                device_id=(dst,),
                device_id_type=pl.DeviceIdType.MESH,
            )
            rdma.start()
        for d in range(1, N_DEV):
            s = (my - d) % N_DEV
            recv = pltpu.make_async_remote_copy(
                src_ref=amax_ref.at[pl.ds(s, 1), :],
                dst_ref=amax_ref.at[pl.ds(s, 1), :],
                send_sem=am_send_sems.at[s],
                recv_sem=am_recv_sems.at[s],
                device_id=(s,),
                device_id_type=pl.DeviceIdType.MESH,
            )
            recv.wait_recv()
        for d in range(1, N_DEV):
            dst = (my + d) % N_DEV
            snd = pltpu.make_async_remote_copy(
                src_ref=amax_ref.at[pl.ds(my, 1), :],
                dst_ref=amax_ref.at[pl.ds(my, 1), :],
                send_sem=am_send_sems.at[dst],
                recv_sem=am_recv_sems.at[dst],
                device_id=(dst,),
                device_id_type=pl.DeviceIdType.MESH,
            )
            snd.wait_send()

        g_amax = jnp.max(amax_ref[...])
        scale = g_amax / 448.0
        q = jnp.minimum(y / scale, 448.0)
        q8 = q.astype(jnp.float8_e4m3fn).astype(jnp.float32)
        out_ref[...] = q8 * scale

    return pl.pallas_call(
        body,
        out_shape=jax.ShapeDtypeStruct((m_per, n), jnp.float32),
        in_specs=[
            pl.BlockSpec(memory_space=pltpu.VMEM),
            pl.BlockSpec(memory_space=pltpu.VMEM),
        ],
        out_specs=pl.BlockSpec(memory_space=pltpu.VMEM),
        scratch_shapes=[
            pltpu.VMEM((N_DEV, m_per, k_shard), jnp.float32),
            pltpu.VMEM((N_DEV, 128), jnp.float32),
            pltpu.SemaphoreType.DMA((N_DEV,)),
            pltpu.SemaphoreType.DMA((N_DEV,)),
            pltpu.SemaphoreType.DMA((N_DEV,)),
            pltpu.SemaphoreType.DMA((N_DEV,)),
        ],
        compiler_params=pltpu.CompilerParams(
            collective_id=0,
            vmem_limit_bytes=100 * 1024 * 1024,
        ),
    )(x, w_mat)


# device time: 57419 ns/iter; 1.2285x vs baseline; 1.0002x over previous
import jax
import jax.numpy as jnp
from jax import lax
from jax.experimental import pallas as pl
from jax.experimental.pallas import tpu as pltpu

N_DEV = 32


def kernel(x, w_mat):
    m_total, k_shard = x.shape
    k_total, n = w_mat.shape
    m_per = m_total // N_DEV
    assert k_total == k_shard * N_DEV

    def body(x_ref, w_ref, out_ref, xg_ref, amax1_ref, amax2_ref,
             a2a_send_sems, a2a_recv_sems,
             am1_send_sems, am1_recv_sems, am2_send_sems, am2_recv_sems):
        my = lax.axis_index("i")

        barrier_sem = pltpu.get_barrier_semaphore()
        for p in range(N_DEV):
            @pl.when(p != my)
            def _():
                pl.semaphore_signal(
                    barrier_sem, inc=1,
                    device_id=(p,), device_id_type=pl.DeviceIdType.MESH,
                )
        pl.semaphore_wait(barrier_sem, N_DEV - 1)

        for d in range(1, N_DEV):
            dst = (my + d) % N_DEV
            rdma = pltpu.make_async_remote_copy(
                src_ref=x_ref.at[pl.ds(dst * m_per, m_per), :],
                dst_ref=xg_ref.at[my],
                send_sem=a2a_send_sems.at[dst],
                recv_sem=a2a_recv_sems.at[my],
                device_id=(dst,),
                device_id_type=pl.DeviceIdType.MESH,
            )
            rdma.start()

        acc = jnp.dot(
            x_ref[pl.ds(my * m_per, m_per), :],
            w_ref[pl.ds(my * k_shard, k_shard), :],
            preferred_element_type=jnp.float32,
        )

        for d in range(1, N_DEV):
            s = (my - d) % N_DEV
            recv = pltpu.make_async_remote_copy(
                src_ref=x_ref.at[pl.ds(0, m_per), :],
                dst_ref=xg_ref.at[s],
                send_sem=a2a_send_sems.at[s],
                recv_sem=a2a_recv_sems.at[s],
                device_id=(s,),
                device_id_type=pl.DeviceIdType.MESH,
            )
            recv.wait_recv()
            acc = acc + jnp.dot(
                xg_ref[s],
                w_ref[pl.ds(s * k_shard, k_shard), :],
                preferred_element_type=jnp.float32,
            )

        for d in range(1, N_DEV):
            dst = (my + d) % N_DEV
            snd = pltpu.make_async_remote_copy(
                src_ref=x_ref.at[pl.ds(dst * m_per, m_per), :],
                dst_ref=xg_ref.at[my],
                send_sem=a2a_send_sems.at[dst],
                recv_sem=a2a_recv_sems.at[my],
                device_id=(dst,),
                device_id_type=pl.DeviceIdType.MESH,
            )
            snd.wait_send()

        y = jnp.maximum(acc, 0.0)
        local_amax = jnp.max(y)

        N_PLANE = 8
        N_Z = N_DEV // N_PLANE
        plane_base = (my // N_PLANE) * N_PLANE
        my_pos = my % N_PLANE
        my_z = my // N_PLANE

        amax1_ref[pl.ds(my_pos, 1), :] = jnp.full((1, 128), local_amax, jnp.float32)
        for j in range(1, N_PLANE):
            dst = plane_base + (my_pos + j) % N_PLANE
            pltpu.make_async_remote_copy(
                src_ref=amax1_ref.at[pl.ds(my_pos, 1), :],
                dst_ref=amax1_ref.at[pl.ds(my_pos, 1), :],
                send_sem=am1_send_sems.at[(my_pos + j) % N_PLANE],
                recv_sem=am1_recv_sems.at[my_pos],
                device_id=(dst,),
                device_id_type=pl.DeviceIdType.MESH,
            ).start()
        for j in range(1, N_PLANE):
            s_pos = (my_pos - j) % N_PLANE
            pltpu.make_async_remote_copy(
                src_ref=amax1_ref.at[pl.ds(s_pos, 1), :],
                dst_ref=amax1_ref.at[pl.ds(s_pos, 1), :],
                send_sem=am1_send_sems.at[s_pos],
                recv_sem=am1_recv_sems.at[s_pos],
                device_id=(plane_base,),
                device_id_type=pl.DeviceIdType.MESH,
            ).wait_recv()
        plane_max = jnp.max(amax1_ref[...])

        amax2_ref[pl.ds(my_z, 1), :] = jnp.full((1, 128), plane_max, jnp.float32)
        for k in range(1, N_Z):
            dst = (my + N_PLANE * k) % N_DEV
            pltpu.make_async_remote_copy(
                src_ref=amax2_ref.at[pl.ds(my_z, 1), :],
                dst_ref=amax2_ref.at[pl.ds(my_z, 1), :],
                send_sem=am2_send_sems.at[(my_z + k) % N_Z],
                recv_sem=am2_recv_sems.at[my_z],
                device_id=(dst,),
                device_id_type=pl.DeviceIdType.MESH,
            ).start()
        for k in range(1, N_Z):
            s_z = (my_z - k) % N_Z
            pltpu.make_async_remote_copy(
                src_ref=amax2_ref.at[pl.ds(s_z, 1), :],
                dst_ref=amax2_ref.at[pl.ds(s_z, 1), :],
                send_sem=am2_send_sems.at[s_z],
                recv_sem=am2_recv_sems.at[s_z],
                device_id=(my_pos,),
                device_id_type=pl.DeviceIdType.MESH,
            ).wait_recv()
        g_amax = jnp.max(amax2_ref[...])

        for j in range(1, N_PLANE):
            pltpu.make_async_remote_copy(
                src_ref=amax1_ref.at[pl.ds(my_pos, 1), :],
                dst_ref=amax1_ref.at[pl.ds(my_pos, 1), :],
                send_sem=am1_send_sems.at[(my_pos + j) % N_PLANE],
                recv_sem=am1_recv_sems.at[my_pos],
                device_id=(plane_base,),
                device_id_type=pl.DeviceIdType.MESH,
            ).wait_send()
        for k in range(1, N_Z):
            pltpu.make_async_remote_copy(
                src_ref=amax2_ref.at[pl.ds(my_z, 1), :],
                dst_ref=amax2_ref.at[pl.ds(my_z, 1), :],
                send_sem=am2_send_sems.at[(my_z + k) % N_Z],
                recv_sem=am2_recv_sems.at[my_z],
                device_id=(my_pos,),
                device_id_type=pl.DeviceIdType.MESH,
            ).wait_send()

        scale = g_amax / 448.0
        q = jnp.minimum(y / scale, 448.0)
        q8 = q.astype(jnp.float8_e4m3fn).astype(jnp.float32)
        out_ref[...] = q8 * scale

    return pl.pallas_call(
        body,
        out_shape=jax.ShapeDtypeStruct((m_per, n), jnp.float32),
        in_specs=[
            pl.BlockSpec(memory_space=pltpu.VMEM),
            pl.BlockSpec(memory_space=pltpu.VMEM),
        ],
        out_specs=pl.BlockSpec(memory_space=pltpu.VMEM),
        scratch_shapes=[
            pltpu.VMEM((N_DEV, m_per, k_shard), jnp.float32),
            pltpu.VMEM((8, 128), jnp.float32),
            pltpu.VMEM((4, 128), jnp.float32),
            pltpu.SemaphoreType.DMA((N_DEV,)),
            pltpu.SemaphoreType.DMA((N_DEV,)),
            pltpu.SemaphoreType.DMA((8,)),
            pltpu.SemaphoreType.DMA((8,)),
            pltpu.SemaphoreType.DMA((4,)),
            pltpu.SemaphoreType.DMA((4,)),
        ],
        compiler_params=pltpu.CompilerParams(
            collective_id=0,
            vmem_limit_bytes=100 * 1024 * 1024,
        ),
    )(x, w_mat)


# device time: 54097 ns/iter; 1.3039x vs baseline; 1.0614x over previous
import jax
import jax.numpy as jnp
from jax import lax
from jax.experimental import pallas as pl
from jax.experimental.pallas import tpu as pltpu

N_DEV = 32


def kernel(x, w_mat):
    m_total, k_shard = x.shape
    k_total, n = w_mat.shape
    m_per = m_total // N_DEV
    assert k_total == k_shard * N_DEV

    def body(x_ref, w_ref, out_ref, xg_ref, amax1_ref, amax2_ref,
             a2a_send_sems, a2a_recv_sems,
             am1_send_sems, am1_recv_sems, am2_send_sems, am2_recv_sems):
        my = lax.axis_index("i")

        with jax.named_scope("phase_barrier"):
            barrier_sem = pltpu.get_barrier_semaphore()
            for nbr in (
                (my + 1) % N_DEV,
                (my - 1) % N_DEV,
            ):
                pl.semaphore_signal(
                    barrier_sem, inc=1,
                    device_id=(nbr,), device_id_type=pl.DeviceIdType.MESH,
                )
            pl.semaphore_wait(barrier_sem, 2)

        with jax.named_scope("phase_issue_sends"):
            for d in range(1, N_DEV):
                dst = (my + d) % N_DEV
                rdma = pltpu.make_async_remote_copy(
                    src_ref=x_ref.at[pl.ds(dst * m_per, m_per), :],
                    dst_ref=xg_ref.at[my],
                    send_sem=a2a_send_sems.at[dst],
                    recv_sem=a2a_recv_sems.at[my],
                    device_id=(dst,),
                    device_id_type=pl.DeviceIdType.MESH,
                )
                rdma.start()

        with jax.named_scope("phase_own_gemm"):
            acc = jnp.dot(
                x_ref[pl.ds(my * m_per, m_per), :],
                w_ref[pl.ds(my * k_shard, k_shard), :],
                preferred_element_type=jnp.float32,
            )

        with jax.named_scope("phase_stream"):
            for d in range(1, N_DEV):
                s = (my - d) % N_DEV
                recv = pltpu.make_async_remote_copy(
                    src_ref=x_ref.at[pl.ds(0, m_per), :],
                    dst_ref=xg_ref.at[s],
                    send_sem=a2a_send_sems.at[s],
                    recv_sem=a2a_recv_sems.at[s],
                    device_id=(s,),
                    device_id_type=pl.DeviceIdType.MESH,
                )
                recv.wait_recv()
                acc = acc + jnp.dot(
                    xg_ref[s],
                    w_ref[pl.ds(s * k_shard, k_shard), :],
                    preferred_element_type=jnp.float32,
                )

        with jax.named_scope("phase_send_waits"):
            for d in range(1, N_DEV):
                dst = (my + d) % N_DEV
                snd = pltpu.make_async_remote_copy(
                    src_ref=x_ref.at[pl.ds(dst * m_per, m_per), :],
                    dst_ref=xg_ref.at[my],
                    send_sem=a2a_send_sems.at[dst],
                    recv_sem=a2a_recv_sems.at[my],
                    device_id=(dst,),
                    device_id_type=pl.DeviceIdType.MESH,
                )
                snd.wait_send()

        with jax.named_scope("phase_relu_amax"):
            y = jnp.maximum(acc, 0.0)
            local_amax = jnp.max(y)

        N_PLANE = 8
        N_Z = N_DEV // N_PLANE
        plane_base = (my // N_PLANE) * N_PLANE
        my_pos = my % N_PLANE
        my_z = my // N_PLANE

        with jax.named_scope("phase_amax1"):
            amax1_ref[pl.ds(my_pos, 1), :] = jnp.full((1, 128), local_amax, jnp.float32)
            for j in range(1, N_PLANE):
                dst = plane_base + (my_pos + j) % N_PLANE
                pltpu.make_async_remote_copy(
                    src_ref=amax1_ref.at[pl.ds(my_pos, 1), :],
                    dst_ref=amax1_ref.at[pl.ds(my_pos, 1), :],
                    send_sem=am1_send_sems.at[(my_pos + j) % N_PLANE],
                    recv_sem=am1_recv_sems.at[my_pos],
                    device_id=(dst,),
                    device_id_type=pl.DeviceIdType.MESH,
                ).start()
            for j in range(1, N_PLANE):
                s_pos = (my_pos - j) % N_PLANE
                pltpu.make_async_remote_copy(
                    src_ref=amax1_ref.at[pl.ds(s_pos, 1), :],
                    dst_ref=amax1_ref.at[pl.ds(s_pos, 1), :],
                    send_sem=am1_send_sems.at[s_pos],
                    recv_sem=am1_recv_sems.at[s_pos],
                    device_id=(plane_base,),
                    device_id_type=pl.DeviceIdType.MESH,
                ).wait_recv()
            plane_max = jnp.max(amax1_ref[...])

        with jax.named_scope("phase_amax2"):
            amax2_ref[pl.ds(my_z, 1), :] = jnp.full((1, 128), plane_max, jnp.float32)
            for k in range(1, N_Z):
                dst = (my + N_PLANE * k) % N_DEV
                pltpu.make_async_remote_copy(
                    src_ref=amax2_ref.at[pl.ds(my_z, 1), :],
                    dst_ref=amax2_ref.at[pl.ds(my_z, 1), :],
                    send_sem=am2_send_sems.at[(my_z + k) % N_Z],
                    recv_sem=am2_recv_sems.at[my_z],
                    device_id=(dst,),
                    device_id_type=pl.DeviceIdType.MESH,
                ).start()
            for k in range(1, N_Z):
                s_z = (my_z - k) % N_Z
                pltpu.make_async_remote_copy(
                    src_ref=amax2_ref.at[pl.ds(s_z, 1), :],
                    dst_ref=amax2_ref.at[pl.ds(s_z, 1), :],
                    send_sem=am2_send_sems.at[s_z],
                    recv_sem=am2_recv_sems.at[s_z],
                    device_id=(my_pos,),
                    device_id_type=pl.DeviceIdType.MESH,
                ).wait_recv()
            g_amax = jnp.max(amax2_ref[...])

        for j in range(1, N_PLANE):
            pltpu.make_async_remote_copy(
                src_ref=amax1_ref.at[pl.ds(my_pos, 1), :],
                dst_ref=amax1_ref.at[pl.ds(my_pos, 1), :],
                send_sem=am1_send_sems.at[(my_pos + j) % N_PLANE],
                recv_sem=am1_recv_sems.at[my_pos],
                device_id=(plane_base,),
                device_id_type=pl.DeviceIdType.MESH,
            ).wait_send()
        for k in range(1, N_Z):
            pltpu.make_async_remote_copy(
                src_ref=amax2_ref.at[pl.ds(my_z, 1), :],
                dst_ref=amax2_ref.at[pl.ds(my_z, 1), :],
                send_sem=am2_send_sems.at[(my_z + k) % N_Z],
                recv_sem=am2_recv_sems.at[my_z],
                device_id=(my_pos,),
                device_id_type=pl.DeviceIdType.MESH,
            ).wait_send()

        with jax.named_scope("phase_quant"):
            scale = g_amax / 448.0
            q = jnp.minimum(y / scale, 448.0)
            q8 = q.astype(jnp.float8_e4m3fn).astype(jnp.float32)
            out_ref[...] = q8 * scale

    return pl.pallas_call(
        body,
        out_shape=jax.ShapeDtypeStruct((m_per, n), jnp.float32),
        in_specs=[
            pl.BlockSpec(memory_space=pltpu.VMEM),
            pl.BlockSpec(memory_space=pltpu.VMEM),
        ],
        out_specs=pl.BlockSpec(memory_space=pltpu.VMEM),
        scratch_shapes=[
            pltpu.VMEM((N_DEV, m_per, k_shard), jnp.float32),
            pltpu.VMEM((8, 128), jnp.float32),
            pltpu.VMEM((4, 128), jnp.float32),
            pltpu.SemaphoreType.DMA((N_DEV,)),
            pltpu.SemaphoreType.DMA((N_DEV,)),
            pltpu.SemaphoreType.DMA((8,)),
            pltpu.SemaphoreType.DMA((8,)),
            pltpu.SemaphoreType.DMA((4,)),
            pltpu.SemaphoreType.DMA((4,)),
        ],
        compiler_params=pltpu.CompilerParams(
            collective_id=0,
            vmem_limit_bytes=100 * 1024 * 1024,
        ),
    )(x, w_mat)


# device time: 54088 ns/iter; 1.3042x vs baseline; 1.0002x over previous
import jax
import jax.numpy as jnp
from jax import lax
from jax.experimental import pallas as pl
from jax.experimental.pallas import tpu as pltpu

N_DEV = 32


def kernel(x, w_mat):
    m_total, k_shard = x.shape
    k_total, n = w_mat.shape
    m_per = m_total // N_DEV
    assert k_total == k_shard * N_DEV

    def body(x_ref, w_ref, out_ref, xg_ref, acc_ref, amax1_ref, amax2_ref,
             a2a_send_sems, a2a_recv_sems,
             am1_send_sems, am1_recv_sems, am2_send_sems, am2_recv_sems):
        my = lax.axis_index("i")

        barrier_sem = pltpu.get_barrier_semaphore()
        for nbr in ((my + 1) % N_DEV, (my - 1) % N_DEV):
            pl.semaphore_signal(
                barrier_sem, inc=1,
                device_id=(nbr,), device_id_type=pl.DeviceIdType.MESH,
            )
        pl.semaphore_wait(barrier_sem, 2)

        for d in range(1, N_DEV):
            dst = (my - d) % N_DEV
            pltpu.make_async_remote_copy(
                src_ref=x_ref.at[pl.ds(dst * m_per, m_per), :],
                dst_ref=xg_ref.at[:, pl.ds(d * k_shard, k_shard)],
                send_sem=a2a_send_sems.at[d],
                recv_sem=a2a_recv_sems.at[d],
                device_id=(dst,),
                device_id_type=pl.DeviceIdType.MESH,
            ).start()

        acc_ref[...] = jnp.dot(
            x_ref[pl.ds(my * m_per, m_per), :],
            w_ref[pl.ds(my * k_shard, k_shard), :],
            preferred_element_type=jnp.float32,
        )

        def wait_slot(d):
            pltpu.make_async_remote_copy(
                src_ref=x_ref.at[pl.ds(0, m_per), :],
                dst_ref=xg_ref.at[:, pl.ds(d * k_shard, k_shard)],
                send_sem=a2a_send_sems.at[d],
                recv_sem=a2a_recv_sems.at[d],
                device_id=(0,),
                device_id_type=pl.DeviceIdType.MESH,
            ).wait_recv()

        G = 4
        for d0 in range(1, N_DEV - 3, G):
            for d in range(d0, d0 + G):
                wait_slot(d)
            base = (my + d0) % N_DEV
            no_wrap = base <= N_DEV - G

            @pl.when(no_wrap)
            def _():
                acc_ref[...] += jnp.dot(
                    xg_ref[:, d0 * k_shard:(d0 + G) * k_shard],
                    w_ref[pl.ds(base * k_shard, G * k_shard), :],
                    preferred_element_type=jnp.float32,
                )

            @pl.when(jnp.logical_not(no_wrap))
            def _():
                for j in range(G):
                    sj = (my + d0 + j) % N_DEV
                    acc_ref[...] += jnp.dot(
                        xg_ref[:, (d0 + j) * k_shard:(d0 + j + 1) * k_shard],
                        w_ref[pl.ds(sj * k_shard, k_shard), :],
                        preferred_element_type=jnp.float32,
                    )

        for d in range(29, N_DEV):
            wait_slot(d)
            s = (my + d) % N_DEV
            acc_ref[...] += jnp.dot(
                xg_ref[:, d * k_shard:(d + 1) * k_shard],
                w_ref[pl.ds(s * k_shard, k_shard), :],
                preferred_element_type=jnp.float32,
            )

        for d in range(1, N_DEV):
            dst = (my - d) % N_DEV
            pltpu.make_async_remote_copy(
                src_ref=x_ref.at[pl.ds(dst * m_per, m_per), :],
                dst_ref=xg_ref.at[:, pl.ds(d * k_shard, k_shard)],
                send_sem=a2a_send_sems.at[d],
                recv_sem=a2a_recv_sems.at[d],
                device_id=(dst,),
                device_id_type=pl.DeviceIdType.MESH,
            ).wait_send()

        y = jnp.maximum(acc_ref[...], 0.0)
        local_amax = jnp.max(y)

        N_PLANE = 8
        N_Z = N_DEV // N_PLANE
        plane_base = (my // N_PLANE) * N_PLANE
        my_pos = my % N_PLANE
        my_z = my // N_PLANE

        amax1_ref[pl.ds(my_pos, 1), :] = jnp.full((1, 128), local_amax, jnp.float32)
        for j in range(1, N_PLANE):
            dst = plane_base + (my_pos + j) % N_PLANE
            pltpu.make_async_remote_copy(
                src_ref=amax1_ref.at[pl.ds(my_pos, 1), :],
                dst_ref=amax1_ref.at[pl.ds(my_pos, 1), :],
                send_sem=am1_send_sems.at[(my_pos + j) % N_PLANE],
                recv_sem=am1_recv_sems.at[my_pos],
                device_id=(dst,),
                device_id_type=pl.DeviceIdType.MESH,
            ).start()
        for j in range(1, N_PLANE):
            s_pos = (my_pos - j) % N_PLANE
            pltpu.make_async_remote_copy(
                src_ref=amax1_ref.at[pl.ds(s_pos, 1), :],
                dst_ref=amax1_ref.at[pl.ds(s_pos, 1), :],
                send_sem=am1_send_sems.at[s_pos],
                recv_sem=am1_recv_sems.at[s_pos],
                device_id=(plane_base,),
                device_id_type=pl.DeviceIdType.MESH,
            ).wait_recv()
        plane_max = jnp.max(amax1_ref[...])

        amax2_ref[pl.ds(my_z, 1), :] = jnp.full((1, 128), plane_max, jnp.float32)
        for k in range(1, N_Z):
            dst = (my + N_PLANE * k) % N_DEV
            pltpu.make_async_remote_copy(
                src_ref=amax2_ref.at[pl.ds(my_z, 1), :],
                dst_ref=amax2_ref.at[pl.ds(my_z, 1), :],
                send_sem=am2_send_sems.at[(my_z + k) % N_Z],
                recv_sem=am2_recv_sems.at[my_z],
                device_id=(dst,),
                device_id_type=pl.DeviceIdType.MESH,
            ).start()
        for k in range(1, N_Z):
            s_z = (my_z - k) % N_Z
            pltpu.make_async_remote_copy(
                src_ref=amax2_ref.at[pl.ds(s_z, 1), :],
                dst_ref=amax2_ref.at[pl.ds(s_z, 1), :],
                send_sem=am2_send_sems.at[s_z],
                recv_sem=am2_recv_sems.at[s_z],
                device_id=(my_pos,),
                device_id_type=pl.DeviceIdType.MESH,
            ).wait_recv()
        g_amax = jnp.max(amax2_ref[...])

        for j in range(1, N_PLANE):
            pltpu.make_async_remote_copy(
                src_ref=amax1_ref.at[pl.ds(my_pos, 1), :],
                dst_ref=amax1_ref.at[pl.ds(my_pos, 1), :],
                send_sem=am1_send_sems.at[(my_pos + j) % N_PLANE],
                recv_sem=am1_recv_sems.at[my_pos],
                device_id=(plane_base,),
                device_id_type=pl.DeviceIdType.MESH,
            ).wait_send()
        for k in range(1, N_Z):
            pltpu.make_async_remote_copy(
                src_ref=amax2_ref.at[pl.ds(my_z, 1), :],
                dst_ref=amax2_ref.at[pl.ds(my_z, 1), :],
                send_sem=am2_send_sems.at[(my_z + k) % N_Z],
                recv_sem=am2_recv_sems.at[my_z],
                device_id=(my_pos,),
                device_id_type=pl.DeviceIdType.MESH,
            ).wait_send()

        inv = 448.0 / g_amax
        scale = g_amax / 448.0
        q = jnp.minimum(y * inv, 448.0)
        q8 = q.astype(jnp.float8_e4m3fn).astype(jnp.float32)
        out_ref[...] = q8 * scale

    return pl.pallas_call(
        body,
        out_shape=jax.ShapeDtypeStruct((m_per, n), jnp.float32),
        in_specs=[
            pl.BlockSpec(memory_space=pltpu.VMEM),
            pl.BlockSpec(memory_space=pltpu.VMEM),
        ],
        out_specs=pl.BlockSpec(memory_space=pltpu.VMEM),
        scratch_shapes=[
            pltpu.VMEM((m_per, k_total), jnp.float32),
            pltpu.VMEM((m_per, n), jnp.float32),
            pltpu.VMEM((8, 128), jnp.float32),
            pltpu.VMEM((4, 128), jnp.float32),
            pltpu.SemaphoreType.DMA((N_DEV,)),
            pltpu.SemaphoreType.DMA((N_DEV,)),
            pltpu.SemaphoreType.DMA((8,)),
            pltpu.SemaphoreType.DMA((8,)),
            pltpu.SemaphoreType.DMA((4,)),
            pltpu.SemaphoreType.DMA((4,)),
        ],
        compiler_params=pltpu.CompilerParams(
            collective_id=0,
            vmem_limit_bytes=100 * 1024 * 1024,
        ),
    )(x, w_mat)
